# baseline (speedup 1.0000x reference)
import functools

import jax
import jax.numpy as jnp
from jax import lax
from jax.experimental import pallas as pl
from jax.experimental.pallas import tpu as pltpu
from jax.experimental.pallas import tpu_sc as plsc

B = 1024
V = 100000
L = 20
D = 32

NC = 2
NS = 16
NW = NC * NS
NTOK = B * L
TOK_PER_W = NTOK // NW

TV = 4096
GRID_V = (V + TV - 1) // TV


TD = 128


def _gather_body(table_hbm, idx_hbm, out_hbm, idx_v, rows_v, sem):
    wid = lax.axis_index("s") * NC + lax.axis_index("c")
    base = wid * TOK_PER_W
    pltpu.sync_copy(idx_hbm.at[pl.ds(base, TOK_PER_W)], idx_v)
    pltpu.async_copy(table_hbm.at[idx_v], rows_v, sem).wait()
    pltpu.sync_copy(rows_v, out_hbm.at[pl.ds(base, TOK_PER_W)])


_sc_gather = pl.kernel(
    _gather_body,
    out_type=jax.ShapeDtypeStruct((NTOK, TD), jnp.float32),
    mesh=plsc.VectorSubcoreMesh(
        core_axis_name="c", subcore_axis_name="s", num_cores=NC, num_subcores=NS
    ),
    scratch_types=[
        pltpu.VMEM((TOK_PER_W,), jnp.int32),
        pltpu.VMEM((TOK_PER_W, TD), jnp.float32),
        pltpu.SemaphoreType.DMA,
    ],
    compiler_params=pltpu.CompilerParams(use_tc_tiling_on_sc=True, needs_layout_passes=False),
)


def _mlp_body(x_ref, w1_ref, b1_ref, w2_ref, b2_ref, out_ref, ht_ref):
    @pl.when(pl.program_id(0) == 0)
    def _():
        ht = lax.dot_general(
            w1_ref[...],
            x_ref[...],
            (((0,), (1,)), ((), ())),
            preferred_element_type=jnp.float32,
        ) + b1_ref[...]
        ht_ref[0:D, :] = ht.astype(jnp.bfloat16)
        ht_ref[D : D + 1, :] = jnp.ones((1, B), jnp.bfloat16)

    w2aug = jnp.concatenate(
        [
            w2_ref[...].astype(jnp.bfloat16),
            b2_ref[...].astype(jnp.bfloat16),
        ],
        axis=0,
    )
    out_ref[...] = lax.dot_general(
        w2aug,
        ht_ref[...],
        (((0,), (0,)), ((), ())),
        preferred_element_type=jnp.float32,
    )


@functools.partial(jax.jit, static_argnames=())
def _mlp(x, W1, b1, W2, b2):
    out_t = pl.pallas_call(
        _mlp_body,
        grid=(GRID_V,),
        in_specs=[
            pl.BlockSpec((B, L * TD), lambda j: (0, 0)),
            pl.BlockSpec((L * TD, D), lambda j: (0, 0)),
            pl.BlockSpec((D, 1), lambda j: (0, 0)),
            pl.BlockSpec((D, TV), lambda j: (0, j)),
            pl.BlockSpec((1, TV), lambda j: (0, j)),
        ],
        out_specs=pl.BlockSpec((TV, B), lambda j: (j, 0)),
        out_shape=jax.ShapeDtypeStruct((V, B), jnp.float32),
        scratch_shapes=[pltpu.VMEM((D + 1, B), jnp.bfloat16)],
    )(x, W1, b1, W2, b2)
    return out_t.T


def kernel(tokens, table, W1, b1, W2, b2):
    idx = tokens.reshape(NTOK)
    tpad = jnp.pad(table, ((0, 0), (0, TD - D)))
    x128 = _sc_gather(tpad, idx).reshape(B, L * TD)
    w1pad = jnp.pad(W1.reshape(L, D, D), ((0, 0), (0, TD - D), (0, 0))).reshape(
        L * TD, D
    )
    return _mlp(x128, w1pad, b1.reshape(D, 1), W2, b2.reshape(1, V))


# speedup vs baseline: 1.0663x; 1.0663x over previous
import functools

import jax
import jax.numpy as jnp
from jax import lax
from jax.experimental import pallas as pl
from jax.experimental.pallas import tpu as pltpu
from jax.experimental.pallas import tpu_sc as plsc

B = 1024
V = 100000
L = 20
D = 32

NC = 2
NS = 16
NW = NC * NS
NTOK = B * L
TOK_PER_W = NTOK // NW

TV = 4096
GRID_V = (V + TV - 1) // TV


def _gather_body(table_hbm, idx_hbm, out_hbm, idx_v, rows_v, sem):
    wid = lax.axis_index("s") * NC + lax.axis_index("c")
    base = wid * TOK_PER_W
    pltpu.sync_copy(idx_hbm.at[pl.ds(base, TOK_PER_W)], idx_v)
    pltpu.async_copy(table_hbm.at[idx_v], rows_v, sem).wait()
    pltpu.sync_copy(rows_v, out_hbm.at[pl.ds(base, TOK_PER_W)])


_sc_gather = pl.kernel(
    _gather_body,
    out_type=jax.ShapeDtypeStruct((NTOK, D), jnp.float32),
    mesh=plsc.VectorSubcoreMesh(
        core_axis_name="c", subcore_axis_name="s", num_cores=NC, num_subcores=NS
    ),
    scratch_types=[
        pltpu.VMEM((TOK_PER_W,), jnp.int32),
        pltpu.VMEM((TOK_PER_W, D), jnp.float32),
        pltpu.SemaphoreType.DMA,
    ],
    compiler_params=pltpu.CompilerParams(use_tc_tiling_on_sc=False),
)


def _mlp_body(x_ref, w1_ref, b1_ref, w2_ref, b2_ref, out_ref, ht_ref):
    @pl.when(pl.program_id(0) == 0)
    def _():
        ht = lax.dot_general(
            w1_ref[...],
            x_ref[...],
            (((0,), (1,)), ((), ())),
            preferred_element_type=jnp.float32,
        ) + b1_ref[...]
        ht_ref[0:D, :] = ht.astype(jnp.bfloat16)
        ht_ref[D : D + 1, :] = jnp.ones((1, B), jnp.bfloat16)

    w2aug = jnp.concatenate(
        [
            w2_ref[...].astype(jnp.bfloat16),
            b2_ref[...].astype(jnp.bfloat16),
        ],
        axis=0,
    )
    out_ref[...] = lax.dot_general(
        w2aug,
        ht_ref[...],
        (((0,), (0,)), ((), ())),
        preferred_element_type=jnp.float32,
    )


@functools.partial(jax.jit, static_argnames=())
def _mlp(x, W1, b1, W2, b2):
    out_t = pl.pallas_call(
        _mlp_body,
        grid=(GRID_V,),
        in_specs=[
            pl.BlockSpec((B, L * D), lambda j: (0, 0)),
            pl.BlockSpec((L * D, D), lambda j: (0, 0)),
            pl.BlockSpec((D, 1), lambda j: (0, 0)),
            pl.BlockSpec((D, TV), lambda j: (0, j)),
            pl.BlockSpec((1, TV), lambda j: (0, j)),
        ],
        out_specs=pl.BlockSpec((TV, B), lambda j: (j, 0)),
        out_shape=jax.ShapeDtypeStruct((V, B), jnp.float32),
        scratch_shapes=[pltpu.VMEM((D + 1, B), jnp.bfloat16)],
    )(x, W1, b1, W2, b2)
    return out_t.T


def kernel(tokens, table, W1, b1, W2, b2):
    idx = tokens.reshape(NTOK)
    x = _sc_gather(table, idx)
    x2 = x.reshape(B, L * D)
    return _mlp(x2, W1, b1.reshape(D, 1), W2, b2.reshape(1, V))
